# baseline (speedup 1.0000x reference)
import functools

import jax
import jax.numpy as jnp
from jax import lax
from jax.experimental import pallas as pl
from jax.experimental.pallas import tpu as pltpu
from jax.experimental.pallas import tpu_sc as plsc

_LANES = 16
_K = 4


def _full16(val):
    return jnp.full((_LANES,), val, dtype=jnp.int32)


def kernel(alpha, mu, D, subj_id):
    B, T, M = alpha.shape
    S, _, C = mu.shape
    P = B * T
    ROWS = S * M
    CC = C * C

    info = plsc.get_sparse_core_info()
    NC, NS = info.num_cores, info.num_subcores
    NW = NC * NS

    nb_max = (P + S * (_K - 1) + _K - 1) // _K
    NB = ((nb_max + NW - 1) // NW) * NW
    BPW = NB // NW

    af = alpha.reshape(P, M)
    sid = subj_id.reshape(P).astype(jnp.int32)
    mur = mu.reshape(ROWS, C)
    Dr = D.reshape(ROWS, CC)

    order = jnp.argsort(sid).astype(jnp.int32)
    ssorted = jnp.take(sid, order)
    run_start = jnp.searchsorted(ssorted, ssorted, side="left").astype(jnp.int32)
    posin = jnp.arange(P, dtype=jnp.int32) - run_start
    counts = jnp.bincount(sid, length=S).astype(jnp.int32)
    nb_s = (counts + _K - 1) // _K
    bbase = jnp.cumsum(nb_s) - nb_s
    gb = jnp.take(bbase, ssorted).astype(jnp.int32) + posin // _K
    slot = posin % _K
    dummy_pair = P + jnp.arange(NB, dtype=jnp.int32) // BPW
    batch_pairs = jnp.broadcast_to(dummy_pair[:, None], (NB, _K))
    batch_pairs = batch_pairs.at[gb, slot].set(order).reshape(NB * _K)
    batch_subj = jnp.zeros((NB,), jnp.int32).at[gb].set(ssorted)
    batch_rows = batch_subj[:, None] * M + jnp.arange(M, dtype=jnp.int32)[None, :]

    mesh = plsc.VectorSubcoreMesh(core_axis_name="c", subcore_axis_name="s")

    @functools.partial(
        pl.kernel,
        out_type=(
            jax.ShapeDtypeStruct((P + NW, C), jnp.float32),
            jax.ShapeDtypeStruct((P + NW, CC), jnp.float32),
        ),
        mesh=mesh,
        compiler_params=pltpu.CompilerParams(use_tc_tiling_on_sc=False,
                                             needs_layout_passes=False),
        scratch_types=[
            pltpu.VMEM((P, M), jnp.float32),
            pltpu.VMEM((BPW, M), jnp.int32),
            pltpu.VMEM((BPW * _K + _LANES,), jnp.int32),
            pltpu.VMEM((M, CC), jnp.float32),
            pltpu.VMEM((M, CC), jnp.float32),
            pltpu.VMEM((M, C), jnp.float32),
            pltpu.VMEM((M, C), jnp.float32),
            pltpu.VMEM((_K, CC), jnp.float32),
            pltpu.VMEM((_K, CC), jnp.float32),
            pltpu.VMEM((_K, C), jnp.float32),
            pltpu.VMEM((_K, C), jnp.float32),
            pltpu.SemaphoreType.DMA,
            pltpu.SemaphoreType.DMA,
            pltpu.SemaphoreType.DMA,
            pltpu.SemaphoreType.DMA,
        ],
    )
    def sc_combine(af_hbm, brows_hbm, bpairs_hbm, mur_hbm, dr_hbm,
                   m_hbm, cov_hbm,
                   alpha_v, brows_v, bpairs_v,
                   dbuf_a, dbuf_b, mubuf_a, mubuf_b,
                   covbuf_a, covbuf_b, moutbuf_a, moutbuf_b,
                   gsem_a, gsem_b, osem_a, osem_b):
        wid = lax.axis_index("s") * NC + lax.axis_index("c")
        gbase = wid * BPW

        pltpu.sync_copy(af_hbm, alpha_v)
        pltpu.sync_copy(brows_hbm.at[pl.ds(gbase, BPW)], brows_v)
        pltpu.sync_copy(bpairs_hbm.at[pl.ds(gbase * _K, BPW * _K)],
                        bpairs_v.at[pl.ds(0, BPW * _K)])

        dbuf = (dbuf_a, dbuf_b)
        mubuf = (mubuf_a, mubuf_b)
        covbuf = (covbuf_a, covbuf_b)
        moutbuf = (moutbuf_a, moutbuf_b)
        gsem = (gsem_a, gsem_b)
        osem = (osem_a, osem_b)

        for par in range(2):
            pltpu.async_copy(dr_hbm.at[brows_v.at[par]], dbuf[par], gsem[par])
            pltpu.async_copy(mur_hbm.at[brows_v.at[par]], mubuf[par], gsem[par])

        def do_batch(g, par):
            db, mb = dbuf[par], mubuf[par]
            cb, mob = covbuf[par], moutbuf[par]
            gs, os = gsem[par], osem[par]

            pltpu.make_async_copy(dr_hbm.at[brows_v.at[g]], db, gs).wait()
            pltpu.make_async_copy(mur_hbm.at[brows_v.at[g]], mb, gs).wait()

            @pl.when(g >= 2)
            def _():
                for k in range(_K):
                    pltpu.make_async_copy(cb.at[k], cov_hbm.at[0], os).wait()
                    pltpu.make_async_copy(mob.at[k], m_hbm.at[0], os).wait()

            a = []
            for k in range(_K):
                pv = plsc.load_gather(bpairs_v, [_full16(g * _K + k)])
                pvc = jnp.minimum(pv, P - 1)
                a.append([plsc.load_gather(alpha_v, [pvc, _full16(m)])
                          for m in range(M)])
            pair_vec = bpairs_v[pl.ds(g * _K, _LANES)]

            def chunk_body(j, carry):
                col = j * _LANES
                d = [db[m, pl.ds(col, _LANES)] for m in range(M)]
                for k in range(_K):
                    acc = d[0] * a[k][0]
                    for m in range(1, M):
                        acc = acc + d[m] * a[k][m]
                    cb[k, pl.ds(col, _LANES)] = acc
                return carry

            lax.fori_loop(0, CC // _LANES, chunk_body, 0)

            for c in range(C // _LANES):
                col = c * _LANES
                dm = [mb[m, pl.ds(col, _LANES)] for m in range(M)]
                for k in range(_K):
                    acc = dm[0] * a[k][0]
                    for m in range(1, M):
                        acc = acc + dm[m] * a[k][m]
                    mob[k, pl.ds(col, _LANES)] = acc

            nxt = jnp.minimum(g + 2, BPW - 1)
            pltpu.async_copy(dr_hbm.at[brows_v.at[nxt]], db, gs)
            pltpu.async_copy(mur_hbm.at[brows_v.at[nxt]], mb, gs)

            for k in range(_K):
                p = pair_vec[k]
                pltpu.async_copy(cb.at[k], cov_hbm.at[p], os)
                pltpu.async_copy(mob.at[k], m_hbm.at[p], os)

        def body(h, carry):
            do_batch(2 * h, 0)
            do_batch(2 * h + 1, 1)
            return carry

        lax.fori_loop(0, BPW // 2, body, 0)

        for par in range(2):
            pltpu.make_async_copy(dr_hbm.at[brows_v.at[0]], dbuf[par],
                                  gsem[par]).wait()
            pltpu.make_async_copy(mur_hbm.at[brows_v.at[0]], mubuf[par],
                                  gsem[par]).wait()
            for k in range(_K):
                pltpu.make_async_copy(covbuf[par].at[k], cov_hbm.at[0],
                                      osem[par]).wait()
                pltpu.make_async_copy(moutbuf[par].at[k], m_hbm.at[0],
                                      osem[par]).wait()

    m2d, cov2d = sc_combine(af, batch_rows, batch_pairs, mur, Dr)
    return (m2d[:P].reshape(B, T, C), cov2d[:P].reshape(B, T, C, C))


# speedup vs baseline: 1.8809x; 1.8809x over previous
import functools

import jax
import jax.numpy as jnp
from jax import lax
from jax.experimental import pallas as pl
from jax.experimental.pallas import tpu as pltpu
from jax.experimental.pallas import tpu_sc as plsc

_LANES = 16
_K = 4


def _full16(val):
    return jnp.full((_LANES,), val, dtype=jnp.int32)


def kernel(alpha, mu, D, subj_id):
    B, T, M = alpha.shape
    S, _, C = mu.shape
    P = B * T
    ROWS = S * M
    CC = C * C

    info = plsc.get_sparse_core_info()
    NC, NS = info.num_cores, info.num_subcores
    NW = NC * NS

    nb_max = (P + S * (_K - 1) + _K - 1) // _K
    NB = ((nb_max + NW - 1) // NW) * NW
    BPW = NB // NW

    af = alpha.reshape(P, M)
    sid = subj_id.reshape(P).astype(jnp.int32)
    mur = mu.reshape(ROWS, C)
    Dr = D.reshape(ROWS, CC)

    iota_p = jnp.arange(P, dtype=jnp.int32)
    _, order = lax.sort((sid, iota_p), num_keys=1)
    onehot = (sid[:, None] == jnp.arange(S, dtype=jnp.int32)[None, :])
    counts = onehot.sum(0).astype(jnp.int32)
    csum_excl = jnp.cumsum(counts) - counts
    nb_s = (counts + _K - 1) // _K
    bbase = jnp.cumsum(nb_s) - nb_s
    bid = jnp.arange(NB, dtype=jnp.int32)
    geb = (bid[:, None] >= bbase[None, :])
    subj_b = geb.sum(1).astype(jnp.int32) - 1
    sb1h = (subj_b[:, None] == jnp.arange(S, dtype=jnp.int32)[None, :])
    c0_b = jnp.where(sb1h, csum_excl[None, :], 0).sum(1).astype(jnp.int32)
    bb_b = jnp.where(sb1h, bbase[None, :], 0).sum(1).astype(jnp.int32)
    n_b = jnp.where(sb1h, counts[None, :], 0).sum(1).astype(jnp.int32)
    q0_b = (bid - bb_b) * _K
    srt0 = (c0_b + q0_b).reshape(NW, BPW)
    nrem = (n_b - q0_b).reshape(NW, BPW)
    brows = (subj_b[:, None] * M
             + jnp.arange(M, dtype=jnp.int32)[None, :]).reshape(NW, BPW, M)

    mesh = plsc.VectorSubcoreMesh(core_axis_name="c", subcore_axis_name="s")

    @functools.partial(
        pl.kernel,
        out_type=(
            jax.ShapeDtypeStruct((P, C), jnp.float32),
            jax.ShapeDtypeStruct((P, CC), jnp.float32),
            jax.ShapeDtypeStruct((NW, C), jnp.float32),
            jax.ShapeDtypeStruct((NW, CC), jnp.float32),
        ),
        mesh=mesh,
        compiler_params=pltpu.CompilerParams(use_tc_tiling_on_sc=False,
                                             needs_layout_passes=False),
        scratch_types=[
            pltpu.VMEM((P, M), jnp.float32),
            pltpu.VMEM((P,), jnp.int32),
            pltpu.VMEM((BPW,), jnp.int32),
            pltpu.VMEM((BPW,), jnp.int32),
            pltpu.VMEM((BPW, M), jnp.int32),
            pltpu.VMEM((M, CC), jnp.float32),
            pltpu.VMEM((M, CC), jnp.float32),
            pltpu.VMEM((M, C), jnp.float32),
            pltpu.VMEM((M, C), jnp.float32),
            pltpu.VMEM((_K, CC), jnp.float32),
            pltpu.VMEM((_K, CC), jnp.float32),
            pltpu.VMEM((_K, C), jnp.float32),
            pltpu.VMEM((_K, C), jnp.float32),
            pltpu.SemaphoreType.DMA,
            pltpu.SemaphoreType.DMA,
            pltpu.SemaphoreType.DMA,
            pltpu.SemaphoreType.DMA,
        ],
    )
    def sc_combine(af_hbm, order_hbm, srt0_hbm, nrem_hbm, brows_hbm,
                   mur_hbm, dr_hbm,
                   m_hbm, cov_hbm, wm_hbm, wcov_hbm,
                   alpha_v, order_v, srt0_v, nrem_v, brows_v,
                   dbuf_a, dbuf_b, mubuf_a, mubuf_b,
                   covbuf_a, covbuf_b, moutbuf_a, moutbuf_b,
                   gsem_a, gsem_b, osem_a, osem_b):
        wid = lax.axis_index("s") * NC + lax.axis_index("c")

        pltpu.sync_copy(af_hbm, alpha_v)
        pltpu.sync_copy(order_hbm, order_v)
        pltpu.sync_copy(srt0_hbm.at[wid], srt0_v)
        pltpu.sync_copy(nrem_hbm.at[wid], nrem_v)
        pltpu.sync_copy(brows_hbm.at[wid], brows_v)

        dbuf = (dbuf_a, dbuf_b)
        mubuf = (mubuf_a, mubuf_b)
        covbuf = (covbuf_a, covbuf_b)
        moutbuf = (moutbuf_a, moutbuf_b)
        gsem = (gsem_a, gsem_b)
        osem = (osem_a, osem_b)

        def issue_gather(g, par):
            idx = brows_v.at[g]
            pltpu.async_copy(dr_hbm.at[idx], dbuf[par], gsem[par])
            pltpu.async_copy(mur_hbm.at[idx], mubuf[par], gsem[par])

        for par in range(2):
            issue_gather(par, par)

        def do_batch(g, par):
            db, mb = dbuf[par], mubuf[par]
            cb, mob = covbuf[par], moutbuf[par]
            gs, os = gsem[par], osem[par]

            idx = brows_v.at[g]
            pltpu.make_async_copy(dr_hbm.at[idx], db, gs).wait()
            pltpu.make_async_copy(mur_hbm.at[idx], mb, gs).wait()

            @pl.when(g >= 2)
            def _():
                for k in range(_K):
                    pltpu.make_async_copy(cb.at[k], cov_hbm.at[0], os).wait()
                    pltpu.make_async_copy(mob.at[k], m_hbm.at[0], os).wait()

            srt0v = plsc.load_gather(srt0_v, [_full16(g)])
            nremv = plsc.load_gather(nrem_v, [_full16(g)])
            n_valid = nremv[0]
            a = []
            pair_ids = []
            for k in range(_K):
                si = jnp.clip(srt0v + k, 0, P - 1)
                pv = plsc.load_gather(order_v, [si])
                pair_ids.append(pv[0])
                a.append([plsc.load_gather(alpha_v, [pv, _full16(m)])
                          for m in range(M)])

            def chunk_body(j, carry):
                col = j * _LANES
                d = [db[m, pl.ds(col, _LANES)] for m in range(M)]
                for k in range(_K):
                    acc = d[0] * a[k][0]
                    for m in range(1, M):
                        acc = acc + d[m] * a[k][m]
                    cb[k, pl.ds(col, _LANES)] = acc
                return carry

            lax.fori_loop(0, CC // _LANES, chunk_body, 0)

            for c in range(C // _LANES):
                col = c * _LANES
                dm = [mb[m, pl.ds(col, _LANES)] for m in range(M)]
                for k in range(_K):
                    acc = dm[0] * a[k][0]
                    for m in range(1, M):
                        acc = acc + dm[m] * a[k][m]
                    mob[k, pl.ds(col, _LANES)] = acc

            issue_gather(jnp.minimum(g + 2, BPW - 1), par)

            for k in range(_K):
                p = pair_ids[k]
                valid = k < n_valid

                @pl.when(valid)
                def _():
                    pltpu.async_copy(cb.at[k], cov_hbm.at[p], os)
                    pltpu.async_copy(mob.at[k], m_hbm.at[p], os)

                @pl.when(jnp.logical_not(valid))
                def _():
                    pltpu.async_copy(cb.at[k], wcov_hbm.at[wid], os)
                    pltpu.async_copy(mob.at[k], wm_hbm.at[wid], os)

        def body(h, carry):
            do_batch(2 * h, 0)
            do_batch(2 * h + 1, 1)
            return carry

        lax.fori_loop(0, BPW // 2, body, 0)

        for par in range(2):
            idx = brows_v.at[0]
            pltpu.make_async_copy(dr_hbm.at[idx], dbuf[par], gsem[par]).wait()
            pltpu.make_async_copy(mur_hbm.at[idx], mubuf[par], gsem[par]).wait()
            for k in range(_K):
                pltpu.make_async_copy(covbuf[par].at[k], cov_hbm.at[0],
                                      osem[par]).wait()
                pltpu.make_async_copy(moutbuf[par].at[k], m_hbm.at[0],
                                      osem[par]).wait()

    m2d, cov2d, _, _ = sc_combine(af, order, srt0, nrem, brows, mur, Dr)
    return (m2d.reshape(B, T, C), cov2d.reshape(B, T, C, C))
